# Initial kernel scaffold; baseline (speedup 1.0000x reference)
import jax
import jax.numpy as jnp
from jax.experimental import pallas as pl


def kernel(x, edge_index, W1, b1, W2, b2):
    raise NotImplementedError("write your pallas kernel here")


# speedup vs baseline: 6.9982x; 6.9982x over previous
import functools

import jax
import jax.numpy as jnp
from jax import lax
from jax.experimental import pallas as pl
from jax.experimental.pallas import tpu as pltpu
from jax.experimental.pallas import tpu_sc as plsc

N = 10000
E = 320000
D = 128
NC = 2
NS = 16
L = 16
NPAD = 10240
CHUNK = 80
EPT = E // (NC * NS)
CPT = EPT // CHUNK
GROUP = 5
NGROUPS = CPT // GROUP
RPT = NPAD // NS


def _sc_aggregate(with_deg: bool):
    out_type = [jax.ShapeDtypeStruct((NC, NPAD, D), jnp.float32)]
    if with_deg:
        out_type.append(jax.ShapeDtypeStruct((NC, 1, NPAD), jnp.float32))

    scratch = {
        "sidx": pltpu.VMEM((GROUP, CHUNK), jnp.int32),
        "didx": pltpu.VMEM((GROUP, CHUNK), jnp.int32),
        "rows": pltpu.VMEM((CHUNK, D), jnp.float32),
        "fbuf": pltpu.VMEM((CHUNK,), jnp.float32),
        "acc": pltpu.VMEM_SHARED((NPAD, D), jnp.float32),
        "dacc": pltpu.VMEM_SHARED((NPAD,), jnp.float32),
        "sem": pltpu.SemaphoreType.DMA,
    }

    mesh = plsc.VectorSubcoreMesh(core_axis_name="c", subcore_axis_name="s")

    def body(x_hbm, src_hbm, dst_hbm, *outs, sidx, didx, rows, fbuf, acc, dacc, sem):
        if with_deg:
            nout, dout = outs
        else:
            (nout,) = outs

        c = lax.axis_index("c")
        s = lax.axis_index("s")
        wid = c * NS + s

        zero16 = jnp.zeros((L,), jnp.float32)

        @pl.loop(0, CHUNK)
        def _zrows(i):
            for j in range(D // L):
                rows[i, pl.ds(j * L, L)] = zero16

        for j in range(CHUNK // L):
            fbuf[pl.ds(j * L, L)] = zero16

        @pl.loop(0, RPT // CHUNK)
        def _zacc(k):
            pltpu.sync_copy(rows, acc.at[pl.ds(s * RPT + k * CHUNK, CHUNK)])
            if with_deg:
                pltpu.sync_copy(fbuf, dacc.at[pl.ds(s * RPT + k * CHUNK, CHUNK)])

        if with_deg:
            one16 = jnp.ones((L,), jnp.float32)
            for j in range(CHUNK // L):
                fbuf[pl.ds(j * L, L)] = one16

        plsc.subcore_barrier()

        group0 = wid * NGROUPS

        @pl.loop(0, NGROUPS)
        def _grp(g):
            pltpu.sync_copy(src_hbm.at[group0 + g], sidx)
            pltpu.sync_copy(dst_hbm.at[group0 + g], didx)
            for j in range(GROUP):
                pltpu.async_copy(x_hbm.at[sidx.at[j]], rows, sem).wait()
                pltpu.sync_copy(rows, acc.at[didx.at[j]], add=True)
                if with_deg:
                    pltpu.sync_copy(fbuf, dacc.at[didx.at[j]], add=True)

        plsc.subcore_barrier()

        pltpu.sync_copy(acc.at[pl.ds(s * RPT, RPT)],
                        nout.at[c, pl.ds(s * RPT, RPT)])
        if with_deg:
            pltpu.sync_copy(dacc.at[pl.ds(s * RPT, RPT)],
                            dout.at[c, 0, pl.ds(s * RPT, RPT)])

    return pl.kernel(
        body,
        out_type=tuple(out_type) if with_deg else out_type[0],
        mesh=mesh,
        scratch_types=scratch,
    )


_sc_agg_deg = _sc_aggregate(with_deg=True)
_sc_agg = _sc_aggregate(with_deg=False)


BR = 2048


def _tc_body(p0, p1, xb, d0, d1, w, b, o):
    ssum = p0[...] + p1[...] + xb[...]
    deg = d0[...] + d1[...]
    inv = 1.0 / (deg + 1.0)
    h = ssum * inv[:, None]
    r = jnp.dot(h, w[...], preferred_element_type=jnp.float32) + b[...][None, :]
    o[...] = jnp.where(r > 0.0, r, jnp.exp(jnp.minimum(r, 0.0)) - 1.0)


def _tc_layer(p0, p1, xp, d0, d1, w, b):
    return pl.pallas_call(
        _tc_body,
        grid=(NPAD // BR,),
        in_specs=[
            pl.BlockSpec((BR, D), lambda i: (i, 0)),
            pl.BlockSpec((BR, D), lambda i: (i, 0)),
            pl.BlockSpec((BR, D), lambda i: (i, 0)),
            pl.BlockSpec((BR,), lambda i: (i,)),
            pl.BlockSpec((BR,), lambda i: (i,)),
            pl.BlockSpec((D, D), lambda i: (0, 0)),
            pl.BlockSpec((D,), lambda i: (0,)),
        ],
        out_specs=pl.BlockSpec((BR, D), lambda i: (i, 0)),
        out_shape=jax.ShapeDtypeStruct((NPAD, D), jnp.float32),
    )(p0, p1, xp, d0, d1, w, b)


def kernel(x, edge_index, W1, b1, W2, b2):
    ngr = E // (GROUP * CHUNK)
    src3d = edge_index[0].reshape(ngr, GROUP, CHUNK)
    dst3d = edge_index[1].reshape(ngr, GROUP, CHUNK)
    xp = jnp.pad(x, ((0, NPAD - N), (0, 0)))

    n1, degp = _sc_agg_deg(xp, src3d, dst3d)
    d0, d1 = degp[0, 0], degp[1, 0]
    h1 = _tc_layer(n1[0], n1[1], xp, d0, d1, W1, b1)
    n2 = _sc_agg(h1, src3d, dst3d)
    h2 = _tc_layer(n2[0], n2[1], h1, d0, d1, W2, b2)
    return h2[:N]


# speedup vs baseline: 10.7351x; 1.5340x over previous
import jax
import jax.numpy as jnp
from jax import lax
from jax.experimental import pallas as pl
from jax.experimental.pallas import tpu as pltpu
from jax.experimental.pallas import tpu_sc as plsc

N = 10000
E = 320000
D = 128
NC = 2
NS = 16
L = 16
NPAD = 10240
CHUNK = 100
EPT = E // (NC * NS)
CPT = EPT // CHUNK
G = 5
NG = CPT // G
SUPER = 4 * G
RPT = NPAD // NS
ZC = 80


def _sc_aggregate(with_deg: bool):
    out_type = [jax.ShapeDtypeStruct((NC, NPAD, D), jnp.float32)]
    if with_deg:
        out_type.append(jax.ShapeDtypeStruct((NC, 1, NPAD), jnp.float32))

    scratch = {
        "sidx": pltpu.VMEM((4, G, CHUNK), jnp.int32),
        "didx": pltpu.VMEM((4, G, CHUNK), jnp.int32),
        "rows": pltpu.VMEM((2, CHUNK, D), jnp.float32),
        "fbuf": pltpu.VMEM((112,), jnp.float32),
        "acc": pltpu.VMEM_SHARED((NPAD, D), jnp.float32),
        "dacc": pltpu.VMEM_SHARED((NPAD,), jnp.float32),
        "gs0": pltpu.SemaphoreType.DMA, "gs1": pltpu.SemaphoreType.DMA,
        "ss0": pltpu.SemaphoreType.DMA, "ss1": pltpu.SemaphoreType.DMA,
        "ds0": pltpu.SemaphoreType.DMA, "ds1": pltpu.SemaphoreType.DMA,
        "is0": pltpu.SemaphoreType.DMA, "is1": pltpu.SemaphoreType.DMA,
        "is2": pltpu.SemaphoreType.DMA, "is3": pltpu.SemaphoreType.DMA,
    }

    mesh = plsc.VectorSubcoreMesh(core_axis_name="c", subcore_axis_name="s")

    def body(x_hbm, src_hbm, dst_hbm, *outs, sidx, didx, rows, fbuf, acc,
             dacc, gs0, gs1, ss0, ss1, ds0, ds1, is0, is1, is2, is3):
        if with_deg:
            nout, dout = outs
        else:
            (nout,) = outs
        gs = (gs0, gs1)
        ss = (ss0, ss1)
        dsm = (ds0, ds1)
        iss = (is0, is1, is2, is3)

        c = lax.axis_index("c")
        s = lax.axis_index("s")
        wid = c * NS + s

        zero16 = jnp.zeros((L,), jnp.float32)

        @pl.loop(0, CHUNK)
        def _zrows(i):
            for j in range(D // L):
                rows[0, i, pl.ds(j * L, L)] = zero16

        for j in range(112 // L):
            fbuf[pl.ds(j * L, L)] = zero16

        zrows = rows.at[0, pl.ds(0, ZC)]
        zdeg = fbuf.at[pl.ds(0, ZC)]

        @pl.loop(0, RPT // ZC)
        def _zacc(k):
            pltpu.sync_copy(zrows, acc.at[pl.ds(s * RPT + k * ZC, ZC)])
            if with_deg:
                pltpu.sync_copy(zdeg, dacc.at[pl.ds(s * RPT + k * ZC, ZC)])

        if with_deg:
            one16 = jnp.ones((L,), jnp.float32)
            for j in range(112 // L):
                fbuf[pl.ds(j * L, L)] = one16
        ones_src = fbuf.at[pl.ds(0, CHUNK)]

        plsc.subcore_barrier()

        def stage(g, slot):
            pltpu.async_copy(src_hbm.at[wid, g], sidx.at[slot], iss[slot])
            pltpu.async_copy(dst_hbm.at[wid, g], didx.at[slot], iss[slot])

        def stage_wait(g, slot):
            pltpu.make_async_copy(src_hbm.at[wid, g], sidx.at[slot],
                                  iss[slot]).wait()
            pltpu.make_async_copy(dst_hbm.at[wid, g], didx.at[slot],
                                  iss[slot]).wait()

        def issue_gather(slot, j, b):
            pltpu.async_copy(x_hbm.at[sidx.at[slot, j]], rows.at[b], gs[b])

        def wait_gather(slot, j, b):
            pltpu.make_async_copy(x_hbm.at[sidx.at[slot, j]], rows.at[b],
                                  gs[b]).wait()

        def issue_scatter(slot, j, b):
            pltpu.async_copy(rows.at[b], acc.at[didx.at[slot, j]], ss[b],
                             add=True)

        def wait_scatter(b):
            pltpu.make_async_copy(rows.at[b], acc.at[didx.at[0, 0]],
                                  ss[b]).wait()

        def issue_deg(slot, j, dslot):
            pltpu.async_copy(ones_src, dacc.at[didx.at[slot, j]], dsm[dslot],
                             add=True)

        def wait_deg(dslot):
            pltpu.make_async_copy(ones_src, dacc.at[didx.at[0, 0]],
                                  dsm[dslot]).wait()

        stage(0, 0)
        stage(1, 1)
        stage(2, 2)
        stage_wait(0, 0)
        issue_gather(0, 0, 0)

        @pl.loop(0, CPT // SUPER)
        def _super(r):
            k0 = r * SUPER
            for t in range(SUPER):
                k = k0 + t
                slot, j, b = (t // G) % 4, t % G, t % 2

                wait_gather(slot, j, b)

                @pl.when(k >= 1)
                def _():
                    wait_scatter(1 - b)

                if with_deg:
                    @pl.when(k >= 2)
                    def _():
                        wait_deg(t % 2)

                if t % G == 2:
                    nslot = ((t // G) + 3) % 4
                    ng = 4 * r + nslot + (4 if nslot < 3 else 0)
                    if nslot < 3:
                        @pl.when(ng < NG)
                        def _():
                            stage(ng, nslot)
                    else:
                        stage(ng, nslot)

                if t % G == G - 1 and t != SUPER - 1:
                    stage_wait(0, (t // G + 1) % 4)
                if t == SUPER - 1:
                    @pl.when(k + 1 < CPT)
                    def _():
                        stage_wait(0, 0)

                @pl.when(k + 1 < CPT)
                def _():
                    nt = (t + 1) % SUPER
                    issue_gather((nt // G) % 4, nt % G, 1 - b)

                issue_scatter(slot, j, b)
                if with_deg:
                    issue_deg(slot, j, t % 2)

        wait_scatter((CPT - 1) % 2)
        if with_deg:
            wait_deg((CPT - 2) % 2)
            wait_deg((CPT - 1) % 2)

        plsc.subcore_barrier()

        pltpu.sync_copy(acc.at[pl.ds(s * RPT, RPT)],
                        nout.at[c, pl.ds(s * RPT, RPT)])
        if with_deg:
            pltpu.sync_copy(dacc.at[pl.ds(s * RPT, RPT)],
                            dout.at[c, 0, pl.ds(s * RPT, RPT)])

    return pl.kernel(
        body,
        out_type=tuple(out_type) if with_deg else out_type[0],
        mesh=mesh,
        scratch_types=scratch,
    )


_sc_agg_deg = _sc_aggregate(with_deg=True)
_sc_agg = _sc_aggregate(with_deg=False)


BR = 2048


def _tc_body(p0, p1, xb, d0, d1, w, b, o):
    ssum = p0[...] + p1[...] + xb[...]
    deg = d0[...] + d1[...]
    inv = 1.0 / (deg + 1.0)
    h = ssum * inv[:, None]
    r = jnp.dot(h, w[...], preferred_element_type=jnp.float32) + b[...][None, :]
    o[...] = jnp.where(r > 0.0, r, jnp.exp(jnp.minimum(r, 0.0)) - 1.0)


def _tc_layer(p0, p1, xp, d0, d1, w, b):
    return pl.pallas_call(
        _tc_body,
        grid=(NPAD // BR,),
        in_specs=[
            pl.BlockSpec((BR, D), lambda i: (i, 0)),
            pl.BlockSpec((BR, D), lambda i: (i, 0)),
            pl.BlockSpec((BR, D), lambda i: (i, 0)),
            pl.BlockSpec((BR,), lambda i: (i,)),
            pl.BlockSpec((BR,), lambda i: (i,)),
            pl.BlockSpec((D, D), lambda i: (0, 0)),
            pl.BlockSpec((D,), lambda i: (0,)),
        ],
        out_specs=pl.BlockSpec((BR, D), lambda i: (i, 0)),
        out_shape=jax.ShapeDtypeStruct((NPAD, D), jnp.float32),
    )(p0, p1, xp, d0, d1, w, b)


def kernel(x, edge_index, W1, b1, W2, b2):
    src3d = edge_index[0].reshape(NC * NS, NG, G, CHUNK)
    dst3d = edge_index[1].reshape(NC * NS, NG, G, CHUNK)
    xp = jnp.pad(x, ((0, NPAD - N), (0, 0)))

    n1, degp = _sc_agg_deg(xp, src3d, dst3d)
    d0, d1 = degp[0, 0], degp[1, 0]
    h1 = _tc_layer(n1[0], n1[1], xp, d0, d1, W1, b1)
    n2 = _sc_agg(h1, src3d, dst3d)
    h2 = _tc_layer(n2[0], n2[1], h1, d0, d1, W2, b2)
    return h2[:N]


# speedup vs baseline: 14.5230x; 1.3529x over previous
import jax
import jax.numpy as jnp
from jax import lax
from jax.experimental import pallas as pl
from jax.experimental.pallas import tpu as pltpu
from jax.experimental.pallas import tpu_sc as plsc

N = 10000
E = 320000
D = 128
NC = 2
NS = 16
L = 16
NPAD = 10240
CHUNK = 100
EPT = E // (NC * NS)
CPT = EPT // CHUNK
G = 5
NG = CPT // G
NBUF = 3
AHEAD = 2
RPT = NPAD // NS
ZC = 80


def _sc_aggregate(with_deg: bool):
    out_type = [jax.ShapeDtypeStruct((NC, NPAD, D), jnp.float32)]
    if with_deg:
        out_type.append(jax.ShapeDtypeStruct((NC, 1, NPAD), jnp.float32))

    scratch = {
        "sidx": pltpu.VMEM((4, G, CHUNK), jnp.int32),
        "didx": pltpu.VMEM((4, G, CHUNK), jnp.int32),
        "rows": pltpu.VMEM((NBUF, CHUNK, D), jnp.float32),
        "fbuf": pltpu.VMEM((112,), jnp.float32),
        "acc": pltpu.VMEM_SHARED((NPAD, D), jnp.float32),
        "dacc": pltpu.VMEM_SHARED((NPAD,), jnp.float32),
        "gsem": pltpu.SemaphoreType.DMA((NBUF,)),
        "ssem": pltpu.SemaphoreType.DMA((NBUF,)),
        "dsem": pltpu.SemaphoreType.DMA((2,)),
        "isem": pltpu.SemaphoreType.DMA((4,)),
    }

    mesh = plsc.VectorSubcoreMesh(core_axis_name="c", subcore_axis_name="s")

    def body(x_hbm, src_hbm, dst_hbm, *outs, sidx, didx, rows, fbuf, acc,
             dacc, gsem, ssem, dsem, isem):
        if with_deg:
            nout, dout = outs
        else:
            (nout,) = outs

        c = lax.axis_index("c")
        s = lax.axis_index("s")
        wid = c * NS + s

        zero16 = jnp.zeros((L,), jnp.float32)

        @pl.loop(0, CHUNK)
        def _zrows(i):
            for j in range(D // L):
                rows[0, i, pl.ds(j * L, L)] = zero16

        for j in range(112 // L):
            fbuf[pl.ds(j * L, L)] = zero16

        zrows = rows.at[0, pl.ds(0, ZC)]
        zdeg = fbuf.at[pl.ds(0, ZC)]

        @pl.loop(0, RPT // ZC)
        def _zacc(k):
            pltpu.sync_copy(zrows, acc.at[pl.ds(s * RPT + k * ZC, ZC)])
            if with_deg:
                pltpu.sync_copy(zdeg, dacc.at[pl.ds(s * RPT + k * ZC, ZC)])

        if with_deg:
            one16 = jnp.ones((L,), jnp.float32)
            for j in range(112 // L):
                fbuf[pl.ds(j * L, L)] = one16
        ones_src = fbuf.at[pl.ds(0, CHUNK)]

        plsc.subcore_barrier()

        def stage(g, slot):
            pltpu.async_copy(src_hbm.at[wid, g], sidx.at[slot], isem.at[slot])
            pltpu.async_copy(dst_hbm.at[wid, g], didx.at[slot], isem.at[slot])

        def stage_wait(slot):
            for _ in range(2):
                pltpu.make_async_copy(src_hbm.at[wid, 0], sidx.at[slot],
                                      isem.at[slot]).wait()

        def issue_gather(k):
            slot, j, b = (k // G) % 4, k % G, k % NBUF
            pltpu.async_copy(x_hbm.at[sidx.at[slot, j]], rows.at[b],
                             gsem.at[b])

        def wait_gather(k):
            b = k % NBUF
            pltpu.make_async_copy(x_hbm.at[sidx.at[0, 0]], rows.at[b],
                                  gsem.at[b]).wait()

        def issue_scatter(k):
            slot, j, b = (k // G) % 4, k % G, k % NBUF
            pltpu.async_copy(rows.at[b], acc.at[didx.at[slot, j]],
                             ssem.at[b], add=True)

        def wait_scatter(k):
            b = k % NBUF
            pltpu.make_async_copy(rows.at[b], acc.at[didx.at[0, 0]],
                                  ssem.at[b]).wait()

        def issue_deg(k):
            slot, j = (k // G) % 4, k % G
            pltpu.async_copy(ones_src, dacc.at[didx.at[slot, j]],
                             dsem.at[k % 2], add=True)

        def wait_deg(k):
            pltpu.make_async_copy(ones_src, dacc.at[didx.at[0, 0]],
                                  dsem.at[k % 2]).wait()

        stage(0, 0)
        stage(1, 1)
        stage(2, 2)
        stage_wait(0)
        for k in range(AHEAD):
            issue_gather(k)

        @pl.loop(0, CPT)
        def _step(k):
            @pl.when(k >= 1)
            def _():
                wait_scatter(k - 1)

            if with_deg:
                @pl.when(k >= 2)
                def _():
                    wait_deg(k - 2)

            g = k // G

            @pl.when((k % G == 2) & (g + 3 < NG))
            def _():
                stage(g + 3, (g + 3) % 4)

            @pl.when((k % G == G - AHEAD) & (k + AHEAD < CPT))
            def _():
                stage_wait((g + 1) % 4)

            @pl.when(k + AHEAD < CPT)
            def _():
                issue_gather(k + AHEAD)

            wait_gather(k)
            issue_scatter(k)
            if with_deg:
                issue_deg(k)

        wait_scatter(CPT - 1)
        if with_deg:
            wait_deg(CPT - 2)
            wait_deg(CPT - 1)

        plsc.subcore_barrier()

        pltpu.sync_copy(acc.at[pl.ds(s * RPT, RPT)],
                        nout.at[c, pl.ds(s * RPT, RPT)])
        if with_deg:
            pltpu.sync_copy(dacc.at[pl.ds(s * RPT, RPT)],
                            dout.at[c, 0, pl.ds(s * RPT, RPT)])

    return pl.kernel(
        body,
        out_type=tuple(out_type) if with_deg else out_type[0],
        mesh=mesh,
        scratch_types=scratch,
    )


_sc_agg_deg = _sc_aggregate(with_deg=True)
_sc_agg = _sc_aggregate(with_deg=False)


BR = 2048


def _tc_body(p0, p1, xb, d0, d1, w, b, o):
    ssum = p0[...] + p1[...] + xb[...]
    deg = d0[...] + d1[...]
    inv = 1.0 / (deg + 1.0)
    h = ssum * inv[:, None]
    r = jnp.dot(h, w[...], preferred_element_type=jnp.float32) + b[...][None, :]
    o[...] = jnp.where(r > 0.0, r, jnp.exp(jnp.minimum(r, 0.0)) - 1.0)


def _tc_layer(p0, p1, xp, d0, d1, w, b):
    return pl.pallas_call(
        _tc_body,
        grid=(NPAD // BR,),
        in_specs=[
            pl.BlockSpec((BR, D), lambda i: (i, 0)),
            pl.BlockSpec((BR, D), lambda i: (i, 0)),
            pl.BlockSpec((BR, D), lambda i: (i, 0)),
            pl.BlockSpec((BR,), lambda i: (i,)),
            pl.BlockSpec((BR,), lambda i: (i,)),
            pl.BlockSpec((D, D), lambda i: (0, 0)),
            pl.BlockSpec((D,), lambda i: (0,)),
        ],
        out_specs=pl.BlockSpec((BR, D), lambda i: (i, 0)),
        out_shape=jax.ShapeDtypeStruct((NPAD, D), jnp.float32),
    )(p0, p1, xp, d0, d1, w, b)


def kernel(x, edge_index, W1, b1, W2, b2):
    src3d = edge_index[0].reshape(NC * NS, NG, G, CHUNK)
    dst3d = edge_index[1].reshape(NC * NS, NG, G, CHUNK)
    xp = jnp.pad(x, ((0, NPAD - N), (0, 0)))

    n1, degp = _sc_agg_deg(xp, src3d, dst3d)
    d0, d1 = degp[0, 0], degp[1, 0]
    h1 = _tc_layer(n1[0], n1[1], xp, d0, d1, W1, b1)
    n2 = _sc_agg(h1, src3d, dst3d)
    h2 = _tc_layer(n2[0], n2[1], h1, d0, d1, W2, b2)
    return h2[:N]


# speedup vs baseline: 14.7060x; 1.0126x over previous
import jax
import jax.numpy as jnp
from jax import lax
from jax.experimental import pallas as pl
from jax.experimental.pallas import tpu as pltpu
from jax.experimental.pallas import tpu_sc as plsc

N = 10000
E = 320000
D = 128
NC = 2
NS = 16
L = 16
NPAD = 10240
CHUNK = 80
EPT = E // (NC * NS)
CPT = EPT // CHUNK
G = 5
NG = CPT // G
NSLOT = 2
NBUF = 4
AHEAD = 3
RPT = NPAD // NS
ZC = 80


def _sc_aggregate(with_deg: bool):
    out_type = [jax.ShapeDtypeStruct((NC, NPAD, D), jnp.float32)]
    if with_deg:
        out_type.append(jax.ShapeDtypeStruct((NC, 1, NPAD), jnp.float32))

    scratch = {
        "sidx": pltpu.VMEM((NSLOT, G, CHUNK), jnp.int32),
        "didx": pltpu.VMEM((NSLOT, G, CHUNK), jnp.int32),
        "rows": pltpu.VMEM((NBUF, CHUNK, D), jnp.float32),
        "fbuf": pltpu.VMEM((112,), jnp.float32),
        "acc": pltpu.VMEM_SHARED((NPAD, D), jnp.float32),
        "dacc": pltpu.VMEM_SHARED((NPAD,), jnp.float32),
        "gsem": pltpu.SemaphoreType.DMA((NBUF,)),
        "ssem": pltpu.SemaphoreType.DMA((NBUF,)),
        "dsem": pltpu.SemaphoreType.DMA((2,)),
        "isem": pltpu.SemaphoreType.DMA((NSLOT,)),
    }

    mesh = plsc.VectorSubcoreMesh(core_axis_name="c", subcore_axis_name="s")

    def body(x_hbm, src_hbm, dst_hbm, *outs, sidx, didx, rows, fbuf, acc,
             dacc, gsem, ssem, dsem, isem):
        if with_deg:
            nout, dout = outs
        else:
            (nout,) = outs

        c = lax.axis_index("c")
        s = lax.axis_index("s")
        wid = c * NS + s

        zero16 = jnp.zeros((L,), jnp.float32)

        @pl.loop(0, CHUNK)
        def _zrows(i):
            for j in range(D // L):
                rows[0, i, pl.ds(j * L, L)] = zero16

        for j in range(112 // L):
            fbuf[pl.ds(j * L, L)] = zero16

        zrows = rows.at[0, pl.ds(0, ZC)]
        zdeg = fbuf.at[pl.ds(0, ZC)]

        @pl.loop(0, RPT // ZC)
        def _zacc(k):
            pltpu.sync_copy(zrows, acc.at[pl.ds(s * RPT + k * ZC, ZC)])
            if with_deg:
                pltpu.sync_copy(zdeg, dacc.at[pl.ds(s * RPT + k * ZC, ZC)])

        if with_deg:
            one16 = jnp.ones((L,), jnp.float32)
            for j in range(112 // L):
                fbuf[pl.ds(j * L, L)] = one16
        ones_src = fbuf.at[pl.ds(0, CHUNK)]

        plsc.subcore_barrier()

        def stage(g, slot):
            pltpu.async_copy(src_hbm.at[wid, g], sidx.at[slot], isem.at[slot])
            pltpu.async_copy(dst_hbm.at[wid, g], didx.at[slot], isem.at[slot])

        def stage_wait(slot):
            for _ in range(2):
                pltpu.make_async_copy(src_hbm.at[wid, 0], sidx.at[slot],
                                      isem.at[slot]).wait()

        def issue_gather(k):
            slot, j, b = (k // G) % NSLOT, k % G, k % NBUF
            pltpu.async_copy(x_hbm.at[sidx.at[slot, j]], rows.at[b],
                             gsem.at[b])

        def wait_gather(k):
            b = k % NBUF
            pltpu.make_async_copy(x_hbm.at[sidx.at[0, 0]], rows.at[b],
                                  gsem.at[b]).wait()

        def issue_scatter(k):
            slot, j, b = (k // G) % NSLOT, k % G, k % NBUF
            pltpu.async_copy(rows.at[b], acc.at[didx.at[slot, j]],
                             ssem.at[b], add=True)

        def wait_scatter(k):
            b = k % NBUF
            pltpu.make_async_copy(rows.at[b], acc.at[didx.at[0, 0]],
                                  ssem.at[b]).wait()

        def issue_deg(k):
            slot, j = (k // G) % NSLOT, k % G
            pltpu.async_copy(ones_src, dacc.at[didx.at[slot, j]],
                             dsem.at[k % 2], add=True)

        def wait_deg(k):
            pltpu.make_async_copy(ones_src, dacc.at[didx.at[0, 0]],
                                  dsem.at[k % 2]).wait()

        stage(0, 0)
        stage(1, 1)
        stage_wait(0)
        for k in range(AHEAD):
            issue_gather(k)

        @pl.loop(0, CPT)
        def _step(k):
            @pl.when(k >= 1)
            def _():
                wait_scatter(k - 1)

            if with_deg:
                @pl.when(k >= 1)
                def _():
                    wait_deg(k - 1)

            g = k // G

            @pl.when((k % G == 0) & (k > 0) & (g + 1 < NG))
            def _():
                stage(g + 1, (g + 1) % NSLOT)

            @pl.when((k % G == G - AHEAD) & (k + AHEAD < CPT))
            def _():
                stage_wait((g + 1) % NSLOT)

            @pl.when(k + AHEAD < CPT)
            def _():
                issue_gather(k + AHEAD)

            wait_gather(k)
            issue_scatter(k)
            if with_deg:
                issue_deg(k)

        wait_scatter(CPT - 1)
        if with_deg:
            wait_deg(CPT - 1)

        plsc.subcore_barrier()

        pltpu.sync_copy(acc.at[pl.ds(s * RPT, RPT)],
                        nout.at[c, pl.ds(s * RPT, RPT)])
        if with_deg:
            pltpu.sync_copy(dacc.at[pl.ds(s * RPT, RPT)],
                            dout.at[c, 0, pl.ds(s * RPT, RPT)])

    return pl.kernel(
        body,
        out_type=tuple(out_type) if with_deg else out_type[0],
        mesh=mesh,
        scratch_types=scratch,
    )


_sc_agg_deg = _sc_aggregate(with_deg=True)
_sc_agg = _sc_aggregate(with_deg=False)


BR = 2048


def _tc_body(p0, p1, xb, d0, d1, w, b, o):
    ssum = p0[...] + p1[...] + xb[...]
    deg = d0[...] + d1[...]
    inv = 1.0 / (deg + 1.0)
    h = ssum * inv[:, None]
    r = jnp.dot(h, w[...], preferred_element_type=jnp.float32) + b[...][None, :]
    o[...] = jnp.where(r > 0.0, r, jnp.exp(jnp.minimum(r, 0.0)) - 1.0)


def _tc_layer(p0, p1, xp, d0, d1, w, b):
    return pl.pallas_call(
        _tc_body,
        grid=(NPAD // BR,),
        in_specs=[
            pl.BlockSpec((BR, D), lambda i: (i, 0)),
            pl.BlockSpec((BR, D), lambda i: (i, 0)),
            pl.BlockSpec((BR, D), lambda i: (i, 0)),
            pl.BlockSpec((BR,), lambda i: (i,)),
            pl.BlockSpec((BR,), lambda i: (i,)),
            pl.BlockSpec((D, D), lambda i: (0, 0)),
            pl.BlockSpec((D,), lambda i: (0,)),
        ],
        out_specs=pl.BlockSpec((BR, D), lambda i: (i, 0)),
        out_shape=jax.ShapeDtypeStruct((NPAD, D), jnp.float32),
    )(p0, p1, xp, d0, d1, w, b)


def kernel(x, edge_index, W1, b1, W2, b2):
    src3d = edge_index[0].reshape(NC * NS, NG, G, CHUNK)
    dst3d = edge_index[1].reshape(NC * NS, NG, G, CHUNK)
    xp = jnp.pad(x, ((0, NPAD - N), (0, 0)))

    n1, degp = _sc_agg_deg(xp, src3d, dst3d)
    d0, d1 = degp[0, 0], degp[1, 0]
    h1 = _tc_layer(n1[0], n1[1], xp, d0, d1, W1, b1)
    n2 = _sc_agg(h1, src3d, dst3d)
    h2 = _tc_layer(n2[0], n2[1], h1, d0, d1, W2, b2)
    return h2[:N]


# speedup vs baseline: 15.3992x; 1.0471x over previous
import jax
import jax.numpy as jnp
from jax import lax
from jax.experimental import pallas as pl
from jax.experimental.pallas import tpu as pltpu
from jax.experimental.pallas import tpu_sc as plsc

N = 10000
E = 320000
D = 128
NC = 2
NS = 16
L = 16
NPAD = 10240
CHUNK = 80
EPT = E // (NC * NS)
CPT = EPT // CHUNK
G = 5
NG = CPT // G
NSLOT = 2
NBUF = 4
AHEAD = 3
RPT = NPAD // NS
ZC = 80


def _sc_aggregate(with_deg: bool):
    out_type = [jax.ShapeDtypeStruct((NC, NPAD, D), jnp.float32)]
    if with_deg:
        out_type.append(jax.ShapeDtypeStruct((NC, 1, NPAD), jnp.float32))

    scratch = {
        "sidx": pltpu.VMEM((NSLOT, G, CHUNK), jnp.int32),
        "didx": pltpu.VMEM((NSLOT, G, CHUNK), jnp.int32),
        "rows": pltpu.VMEM((NBUF, CHUNK, D), jnp.float32),
        "fbuf": pltpu.VMEM((112,), jnp.float32),
        "acc": pltpu.VMEM_SHARED((NPAD, D), jnp.float32),
        "dacc": pltpu.VMEM_SHARED((NPAD,), jnp.float32),
        "gsem": pltpu.SemaphoreType.DMA((NBUF,)),
        "ssem": pltpu.SemaphoreType.DMA((NBUF,)),
        "dsem": pltpu.SemaphoreType.DMA((2,)),
        "isem": pltpu.SemaphoreType.DMA((NSLOT,)),
    }

    mesh = plsc.VectorSubcoreMesh(core_axis_name="c", subcore_axis_name="s")

    def body(x_hbm, src_hbm, dst_hbm, *outs, sidx, didx, rows, fbuf, acc,
             dacc, gsem, ssem, dsem, isem):
        if with_deg:
            nout, dout = outs
        else:
            (nout,) = outs

        c = lax.axis_index("c")
        s = lax.axis_index("s")
        wid = c * NS + s

        def stage(g, slot):
            pltpu.async_copy(src_hbm.at[wid, g], sidx.at[slot], isem.at[slot])
            pltpu.async_copy(dst_hbm.at[wid, g], didx.at[slot], isem.at[slot])

        def stage_wait(slot):
            for _ in range(2):
                pltpu.make_async_copy(src_hbm.at[wid, 0], sidx.at[slot],
                                      isem.at[slot]).wait()

        def issue_gather(k):
            slot, j, b = (k // G) % NSLOT, k % G, k % NBUF
            pltpu.async_copy(x_hbm.at[sidx.at[slot, j]], rows.at[b],
                             gsem.at[b])

        def wait_gather(k):
            b = k % NBUF
            pltpu.make_async_copy(x_hbm.at[sidx.at[0, 0]], rows.at[b],
                                  gsem.at[b]).wait()

        def issue_scatter(k):
            slot, j, b = (k // G) % NSLOT, k % G, k % NBUF
            pltpu.async_copy(rows.at[b], acc.at[didx.at[slot, j]],
                             ssem.at[b], add=True)

        def wait_scatter(k):
            b = k % NBUF
            pltpu.make_async_copy(rows.at[b], acc.at[didx.at[0, 0]],
                                  ssem.at[b]).wait()

        def issue_deg(k):
            slot, j = (k // G) % NSLOT, k % G
            pltpu.async_copy(ones_src, dacc.at[didx.at[slot, j]],
                             dsem.at[k % 2], add=True)

        def wait_deg(k):
            pltpu.make_async_copy(ones_src, dacc.at[didx.at[0, 0]],
                                  dsem.at[k % 2]).wait()

        zero16 = jnp.zeros((L,), jnp.float32)
        for j in range(112 // L):
            fbuf[pl.ds(j * L, L)] = zero16

        stage(0, 0)
        stage(1, 1)

        @pl.loop(0, CHUNK)
        def _zrows(i):
            for j in range(D // L):
                rows[0, i, pl.ds(j * L, L)] = zero16

        zrows = rows.at[0, pl.ds(0, ZC)]
        zdeg = fbuf.at[pl.ds(0, ZC)]
        for k in range(RPT // ZC):
            pltpu.async_copy(zrows, acc.at[pl.ds(s * RPT + k * ZC, ZC)],
                             ssem.at[k % NBUF])
            if with_deg:
                pltpu.async_copy(zdeg, dacc.at[pl.ds(s * RPT + k * ZC, ZC)],
                                 dsem.at[k % 2])
        for k in range(RPT // ZC):
            pltpu.make_async_copy(zrows, acc.at[pl.ds(0, ZC)],
                                  ssem.at[k % NBUF]).wait()
            if with_deg:
                pltpu.make_async_copy(zdeg, dacc.at[pl.ds(0, ZC)],
                                      dsem.at[k % 2]).wait()

        if with_deg:
            one16 = jnp.ones((L,), jnp.float32)
            for j in range(112 // L):
                fbuf[pl.ds(j * L, L)] = one16
        ones_src = fbuf.at[pl.ds(0, CHUNK)]

        stage_wait(0)
        for k in range(AHEAD):
            issue_gather(k)

        plsc.subcore_barrier()

        @pl.loop(0, CPT)
        def _step(k):
            @pl.when(k >= 1)
            def _():
                wait_scatter(k - 1)

            if with_deg:
                @pl.when(k >= 1)
                def _():
                    wait_deg(k - 1)

            g = k // G

            @pl.when((k % G == 0) & (k > 0) & (g + 1 < NG))
            def _():
                stage(g + 1, (g + 1) % NSLOT)

            @pl.when((k % G == G - AHEAD) & (k + AHEAD < CPT))
            def _():
                stage_wait((g + 1) % NSLOT)

            @pl.when(k + AHEAD < CPT)
            def _():
                issue_gather(k + AHEAD)

            wait_gather(k)
            issue_scatter(k)
            if with_deg:
                issue_deg(k)

        wait_scatter(CPT - 1)
        if with_deg:
            wait_deg(CPT - 1)

        plsc.subcore_barrier()

        pltpu.sync_copy(acc.at[pl.ds(s * RPT, RPT)],
                        nout.at[c, pl.ds(s * RPT, RPT)])
        if with_deg:
            pltpu.sync_copy(dacc.at[pl.ds(s * RPT, RPT)],
                            dout.at[c, 0, pl.ds(s * RPT, RPT)])

    return pl.kernel(
        body,
        out_type=tuple(out_type) if with_deg else out_type[0],
        mesh=mesh,
        scratch_types=scratch,
    )


_sc_agg_deg = _sc_aggregate(with_deg=True)
_sc_agg = _sc_aggregate(with_deg=False)


BR = 2048


def _tc_body(p0, p1, xb, d0, d1, w, b, o):
    ssum = p0[...] + p1[...] + xb[...]
    deg = d0[...] + d1[...]
    inv = 1.0 / (deg + 1.0)
    h = ssum * inv[:, None]
    r = jnp.dot(h, w[...], preferred_element_type=jnp.float32) + b[...][None, :]
    o[...] = jnp.where(r > 0.0, r, jnp.exp(jnp.minimum(r, 0.0)) - 1.0)


def _tc_layer(p0, p1, xs, d0, d1, w, b):
    return pl.pallas_call(
        _tc_body,
        grid=(NPAD // BR,),
        in_specs=[
            pl.BlockSpec((BR, D), lambda i: (i, 0)),
            pl.BlockSpec((BR, D), lambda i: (i, 0)),
            pl.BlockSpec((BR, D), lambda i: (i, 0)),
            pl.BlockSpec((BR,), lambda i: (i,)),
            pl.BlockSpec((BR,), lambda i: (i,)),
            pl.BlockSpec((D, D), lambda i: (0, 0)),
            pl.BlockSpec((D,), lambda i: (0,)),
        ],
        out_specs=pl.BlockSpec((BR, D), lambda i: (i, 0)),
        out_shape=jax.ShapeDtypeStruct((N, D), jnp.float32),
    )(p0, p1, xs, d0, d1, w, b)


def kernel(x, edge_index, W1, b1, W2, b2):
    src4d = edge_index[0].reshape(NC * NS, NG, G, CHUNK)
    dst4d = edge_index[1].reshape(NC * NS, NG, G, CHUNK)

    n1, degp = _sc_agg_deg(x, src4d, dst4d)
    d0, d1 = degp[0, 0], degp[1, 0]
    h1 = _tc_layer(n1[0], n1[1], x, d0, d1, W1, b1)
    n2 = _sc_agg(h1, src4d, dst4d)
    h2 = _tc_layer(n2[0], n2[1], h1, d0, d1, W2, b2)
    return h2


# speedup vs baseline: 16.3830x; 1.0639x over previous
import jax
import jax.numpy as jnp
from jax import lax
from jax.experimental import pallas as pl
from jax.experimental.pallas import tpu as pltpu
from jax.experimental.pallas import tpu_sc as plsc

N = 10000
E = 320000
D = 128
NC = 2
NS = 16
L = 16
NPAD = 10240
CHUNK = 80
EPT = E // (NC * NS)
CPT = EPT // CHUNK
G = 5
NG = CPT // G
NSLOT = 2
NBUF = 4
AHEAD = 3
RPT = NPAD // NS
ZC = 80


def _sc_aggregate(with_deg: bool):
    out_type = [jax.ShapeDtypeStruct((NC, NPAD, D), jnp.float32)]
    if with_deg:
        out_type.append(jax.ShapeDtypeStruct((NC, 1, NPAD), jnp.float32))

    scratch = {
        "sidx": pltpu.VMEM((NSLOT * G * CHUNK,), jnp.int32),
        "didx": pltpu.VMEM((NSLOT, G, CHUNK), jnp.int32),
        "rows": pltpu.VMEM((NBUF, CHUNK, D), jnp.float32),
        "fbuf": pltpu.VMEM((112,), jnp.float32),
        "acc": pltpu.VMEM_SHARED((NPAD, D), jnp.float32),
        "dacc": pltpu.VMEM_SHARED((NPAD,), jnp.float32),
        "gsem": pltpu.SemaphoreType.DMA((NBUF,)),
        "ssem": pltpu.SemaphoreType.DMA((NBUF,)),
        "dsem": pltpu.SemaphoreType.DMA((2,)),
        "isem": pltpu.SemaphoreType.DMA((NSLOT,)),
    }

    mesh = plsc.VectorSubcoreMesh(core_axis_name="c", subcore_axis_name="s")

    def body(x_hbm, src_hbm, dst_hbm, *outs, sidx, didx, rows, fbuf, acc,
             dacc, gsem, ssem, dsem, isem):
        if with_deg:
            nout, dout = outs
        else:
            (nout,) = outs

        c = lax.axis_index("c")
        s = lax.axis_index("s")
        wid = c * NS + s

        def stage(g, slot):
            pltpu.async_copy(
                src_hbm.at[pl.ds(wid * EPT + g * (G * CHUNK), G * CHUNK)],
                sidx.at[pl.ds(slot * (G * CHUNK), G * CHUNK)], isem.at[slot])
            pltpu.async_copy(dst_hbm.at[wid, g], didx.at[slot], isem.at[slot])

        def stage_wait(slot):
            pltpu.make_async_copy(src_hbm.at[pl.ds(0, G * CHUNK)],
                                  sidx.at[pl.ds(0, G * CHUNK)],
                                  isem.at[slot]).wait()
            pltpu.make_async_copy(dst_hbm.at[wid, 0], didx.at[slot],
                                  isem.at[slot]).wait()

        def issue_gather(k):
            slot, j, b = (k // G) % NSLOT, k % G, k % NBUF
            pltpu.async_copy(
                x_hbm.at[sidx.at[pl.ds(slot * (G * CHUNK) + j * CHUNK,
                                       CHUNK)]],
                rows.at[b], gsem.at[b])

        def wait_gather(k):
            b = k % NBUF
            pltpu.make_async_copy(x_hbm.at[sidx.at[pl.ds(0, CHUNK)]],
                                  rows.at[b], gsem.at[b]).wait()

        def issue_scatter(k):
            slot, j, b = (k // G) % NSLOT, k % G, k % NBUF
            pltpu.async_copy(rows.at[b], acc.at[didx.at[slot, j]],
                             ssem.at[b], add=True)

        def wait_scatter(k):
            b = k % NBUF
            pltpu.make_async_copy(rows.at[b], acc.at[didx.at[0, 0]],
                                  ssem.at[b]).wait()

        def issue_deg(k):
            slot, j = (k // G) % NSLOT, k % G
            pltpu.async_copy(ones_src, dacc.at[didx.at[slot, j]],
                             dsem.at[k % 2], add=True)

        def wait_deg(k):
            pltpu.make_async_copy(ones_src, dacc.at[didx.at[0, 0]],
                                  dsem.at[k % 2]).wait()

        zero16 = jnp.zeros((L,), jnp.float32)
        for j in range(112 // L):
            fbuf[pl.ds(j * L, L)] = zero16

        stage(0, 0)
        stage(1, 1)

        @pl.loop(0, CHUNK)
        def _zrows(i):
            for j in range(D // L):
                rows[0, i, pl.ds(j * L, L)] = zero16

        zrows = rows.at[0, pl.ds(0, ZC)]
        zdeg = fbuf.at[pl.ds(0, ZC)]
        for k in range(RPT // ZC):
            pltpu.async_copy(zrows, acc.at[pl.ds(s * RPT + k * ZC, ZC)],
                             ssem.at[k % NBUF])
            if with_deg:
                pltpu.async_copy(zdeg, dacc.at[pl.ds(s * RPT + k * ZC, ZC)],
                                 dsem.at[k % 2])
        for k in range(RPT // ZC):
            pltpu.make_async_copy(zrows, acc.at[pl.ds(0, ZC)],
                                  ssem.at[k % NBUF]).wait()
            if with_deg:
                pltpu.make_async_copy(zdeg, dacc.at[pl.ds(0, ZC)],
                                      dsem.at[k % 2]).wait()

        if with_deg:
            one16 = jnp.ones((L,), jnp.float32)
            for j in range(112 // L):
                fbuf[pl.ds(j * L, L)] = one16
        ones_src = fbuf.at[pl.ds(0, CHUNK)]

        stage_wait(0)
        for k in range(AHEAD):
            issue_gather(k)

        plsc.subcore_barrier()

        @pl.loop(0, CPT)
        def _step(k):
            @pl.when(k >= 1)
            def _():
                wait_scatter(k - 1)

            if with_deg:
                @pl.when(k >= 1)
                def _():
                    wait_deg(k - 1)

            g = k // G

            @pl.when((k % G == 0) & (k > 0) & (g + 1 < NG))
            def _():
                stage(g + 1, (g + 1) % NSLOT)

            @pl.when((k % G == G - AHEAD) & (k + AHEAD < CPT))
            def _():
                stage_wait((g + 1) % NSLOT)

            @pl.when(k + AHEAD < CPT)
            def _():
                issue_gather(k + AHEAD)

            wait_gather(k)
            issue_scatter(k)
            if with_deg:
                issue_deg(k)

        wait_scatter(CPT - 1)
        if with_deg:
            wait_deg(CPT - 1)

        plsc.subcore_barrier()

        pltpu.sync_copy(acc.at[pl.ds(s * RPT, RPT)],
                        nout.at[c, pl.ds(s * RPT, RPT)])
        if with_deg:
            pltpu.sync_copy(dacc.at[pl.ds(s * RPT, RPT)],
                            dout.at[c, 0, pl.ds(s * RPT, RPT)])

    return pl.kernel(
        body,
        out_type=tuple(out_type) if with_deg else out_type[0],
        mesh=mesh,
        scratch_types=scratch,
    )


_sc_agg_deg = _sc_aggregate(with_deg=True)
_sc_agg = _sc_aggregate(with_deg=False)


BR = 2048


def _tc_body(pp, dd, xb, w, b, o):
    ssum = pp[0] + pp[1] + xb[...]
    deg = dd[0, 0] + dd[1, 0]
    inv = 1.0 / (deg + 1.0)
    h = ssum * inv[:, None]
    r = jnp.dot(h, w[...], preferred_element_type=jnp.float32) + b[...][None, :]
    o[...] = jnp.where(r > 0.0, r, jnp.exp(jnp.minimum(r, 0.0)) - 1.0)


def _tc_layer(np2, degp, xs, w, b):
    return pl.pallas_call(
        _tc_body,
        grid=(NPAD // BR,),
        in_specs=[
            pl.BlockSpec((NC, BR, D), lambda i: (0, i, 0)),
            pl.BlockSpec((NC, 1, BR), lambda i: (0, 0, i)),
            pl.BlockSpec((BR, D), lambda i: (i, 0)),
            pl.BlockSpec((D, D), lambda i: (0, 0)),
            pl.BlockSpec((D,), lambda i: (0,)),
        ],
        out_specs=pl.BlockSpec((BR, D), lambda i: (i, 0)),
        out_shape=jax.ShapeDtypeStruct((N, D), jnp.float32),
    )(np2, degp, xs, w, b)


def kernel(x, edge_index, W1, b1, W2, b2):
    src_flat = edge_index[0]
    dst4d = edge_index[1].reshape(NC * NS, NG, G, CHUNK)

    n1, degp = _sc_agg_deg(x, src_flat, dst4d)
    h1 = _tc_layer(n1, degp, x, W1, b1)
    n2 = _sc_agg(h1, src_flat, dst4d)
    h2 = _tc_layer(n2, degp, h1, W2, b2)
    return h2


# speedup vs baseline: 16.6667x; 1.0173x over previous
import jax
import jax.numpy as jnp
from jax import lax
from jax.experimental import pallas as pl
from jax.experimental.pallas import tpu as pltpu
from jax.experimental.pallas import tpu_sc as plsc

N = 10000
E = 320000
D = 128
NC = 2
NS = 16
L = 16
NPAD = 10240
CHUNK = 80
EPT = E // (NC * NS)
CPT = EPT // CHUNK
G = 5
NG = CPT // G
NSLOT = 2
NBUF = 4
AHEAD = 3
RPT = NPAD // NS
ZC = 80


def _sc_aggregate(with_deg: bool):
    out_type = [jax.ShapeDtypeStruct((NC, NPAD, D), jnp.float32)]
    if with_deg:
        out_type.append(jax.ShapeDtypeStruct((NC, 1, NPAD), jnp.float32))

    scratch = {
        "sidx": pltpu.VMEM((NSLOT * G * CHUNK,), jnp.int32),
        "didx": pltpu.VMEM((NSLOT * G * CHUNK,), jnp.int32),
        "rows": pltpu.VMEM((NBUF, CHUNK, D), jnp.float32),
        "fbuf": pltpu.VMEM((112,), jnp.float32),
        "acc": pltpu.VMEM_SHARED((NPAD, D), jnp.float32),
        "dacc": pltpu.VMEM_SHARED((NPAD,), jnp.float32),
        "gsem": pltpu.SemaphoreType.DMA((NBUF,)),
        "ssem": pltpu.SemaphoreType.DMA((NBUF,)),
        "dsem": pltpu.SemaphoreType.DMA((2,)),
        "isem": pltpu.SemaphoreType.DMA((NSLOT,)),
    }

    mesh = plsc.VectorSubcoreMesh(core_axis_name="c", subcore_axis_name="s")

    def body(x_hbm, src_hbm, dst_hbm, *outs, sidx, didx, rows, fbuf, acc,
             dacc, gsem, ssem, dsem, isem):
        if with_deg:
            nout, dout = outs
        else:
            (nout,) = outs

        c = lax.axis_index("c")
        s = lax.axis_index("s")
        wid = c * NS + s

        def stage(g, slot):
            base = wid * EPT + g * (G * CHUNK)
            sbase = slot * (G * CHUNK)
            pltpu.async_copy(src_hbm.at[pl.ds(base, G * CHUNK)],
                             sidx.at[pl.ds(sbase, G * CHUNK)], isem.at[slot])
            pltpu.async_copy(dst_hbm.at[pl.ds(base, G * CHUNK)],
                             didx.at[pl.ds(sbase, G * CHUNK)], isem.at[slot])

        def stage_wait(slot):
            for _ in range(2):
                pltpu.make_async_copy(src_hbm.at[pl.ds(0, G * CHUNK)],
                                      sidx.at[pl.ds(0, G * CHUNK)],
                                      isem.at[slot]).wait()

        def issue_gather(k):
            slot, j, b = (k // G) % NSLOT, k % G, k % NBUF
            pltpu.async_copy(
                x_hbm.at[sidx.at[pl.ds(slot * (G * CHUNK) + j * CHUNK,
                                       CHUNK)]],
                rows.at[b], gsem.at[b])

        def wait_gather(k):
            b = k % NBUF
            pltpu.make_async_copy(x_hbm.at[sidx.at[pl.ds(0, CHUNK)]],
                                  rows.at[b], gsem.at[b]).wait()

        def didx_ref(k):
            slot, j = (k // G) % NSLOT, k % G
            return didx.at[pl.ds(slot * (G * CHUNK) + j * CHUNK, CHUNK)]

        def issue_scatter(k):
            b = k % NBUF
            pltpu.async_copy(rows.at[b], acc.at[didx_ref(k)],
                             ssem.at[b], add=True)

        def wait_scatter(k):
            b = k % NBUF
            pltpu.make_async_copy(rows.at[b], acc.at[didx_ref(0)],
                                  ssem.at[b]).wait()

        def issue_deg(k):
            pltpu.async_copy(ones_src, dacc.at[didx_ref(k)],
                             dsem.at[k % 2], add=True)

        def wait_deg(k):
            pltpu.make_async_copy(ones_src, dacc.at[didx_ref(0)],
                                  dsem.at[k % 2]).wait()

        zero16 = jnp.zeros((L,), jnp.float32)
        for j in range(112 // L):
            fbuf[pl.ds(j * L, L)] = zero16

        stage(0, 0)
        stage(1, 1)

        @pl.loop(0, CHUNK)
        def _zrows(i):
            for j in range(D // L):
                rows[0, i, pl.ds(j * L, L)] = zero16

        zrows = rows.at[0, pl.ds(0, ZC)]
        zdeg = fbuf.at[pl.ds(0, ZC)]
        for k in range(RPT // ZC):
            pltpu.async_copy(zrows, acc.at[pl.ds(s * RPT + k * ZC, ZC)],
                             ssem.at[k % NBUF])
            if with_deg:
                pltpu.async_copy(zdeg, dacc.at[pl.ds(s * RPT + k * ZC, ZC)],
                                 dsem.at[k % 2])
        for k in range(RPT // ZC):
            pltpu.make_async_copy(zrows, acc.at[pl.ds(0, ZC)],
                                  ssem.at[k % NBUF]).wait()
            if with_deg:
                pltpu.make_async_copy(zdeg, dacc.at[pl.ds(0, ZC)],
                                      dsem.at[k % 2]).wait()

        if with_deg:
            one16 = jnp.ones((L,), jnp.float32)
            for j in range(112 // L):
                fbuf[pl.ds(j * L, L)] = one16
        ones_src = fbuf.at[pl.ds(0, CHUNK)]

        stage_wait(0)
        for k in range(AHEAD):
            issue_gather(k)

        plsc.subcore_barrier()

        @pl.loop(0, CPT)
        def _step(k):
            @pl.when(k >= 1)
            def _():
                wait_scatter(k - 1)

            if with_deg:
                @pl.when(k >= 1)
                def _():
                    wait_deg(k - 1)

            g = k // G

            @pl.when((k % G == 0) & (k > 0) & (g + 1 < NG))
            def _():
                stage(g + 1, (g + 1) % NSLOT)

            @pl.when((k % G == G - AHEAD) & (k + AHEAD < CPT))
            def _():
                stage_wait((g + 1) % NSLOT)

            @pl.when(k + AHEAD < CPT)
            def _():
                issue_gather(k + AHEAD)

            wait_gather(k)
            issue_scatter(k)
            if with_deg:
                issue_deg(k)

        wait_scatter(CPT - 1)
        if with_deg:
            wait_deg(CPT - 1)

        plsc.subcore_barrier()

        pltpu.sync_copy(acc.at[pl.ds(s * RPT, RPT)],
                        nout.at[c, pl.ds(s * RPT, RPT)])
        if with_deg:
            pltpu.sync_copy(dacc.at[pl.ds(s * RPT, RPT)],
                            dout.at[c, 0, pl.ds(s * RPT, RPT)])

    return pl.kernel(
        body,
        out_type=tuple(out_type) if with_deg else out_type[0],
        mesh=mesh,
        scratch_types=scratch,
    )


_sc_agg_deg = _sc_aggregate(with_deg=True)
_sc_agg = _sc_aggregate(with_deg=False)


BR = 2048


def _tc_body(pp, dd, xb, w, b, o):
    ssum = pp[0] + pp[1] + xb[...]
    deg = dd[0, 0] + dd[1, 0]
    inv = 1.0 / (deg + 1.0)
    h = ssum * inv[:, None]
    r = jnp.dot(h, w[...], preferred_element_type=jnp.float32) + b[...][None, :]
    o[...] = jnp.where(r > 0.0, r, jnp.exp(jnp.minimum(r, 0.0)) - 1.0)


def _tc_layer(np2, degp, xs, w, b):
    return pl.pallas_call(
        _tc_body,
        grid=(NPAD // BR,),
        in_specs=[
            pl.BlockSpec((NC, BR, D), lambda i: (0, i, 0)),
            pl.BlockSpec((NC, 1, BR), lambda i: (0, 0, i)),
            pl.BlockSpec((BR, D), lambda i: (i, 0)),
            pl.BlockSpec((D, D), lambda i: (0, 0)),
            pl.BlockSpec((D,), lambda i: (0,)),
        ],
        out_specs=pl.BlockSpec((BR, D), lambda i: (i, 0)),
        out_shape=jax.ShapeDtypeStruct((N, D), jnp.float32),
    )(np2, degp, xs, w, b)


def kernel(x, edge_index, W1, b1, W2, b2):
    src_flat = edge_index[0]
    dst_flat = edge_index[1]

    n1, degp = _sc_agg_deg(x, src_flat, dst_flat)
    h1 = _tc_layer(n1, degp, x, W1, b1)
    n2 = _sc_agg(h1, src_flat, dst_flat)
    h2 = _tc_layer(n2, degp, h1, W2, b2)
    return h2
